# baseline (speedup 1.0000x reference)
import functools

import jax
import jax.numpy as jnp
from jax import lax
from jax.experimental import pallas as pl
from jax.experimental.pallas import tpu as pltpu
from jax.experimental.pallas import tpu_sc as plsc

_NC = 2
_NS = 16
_L = 16


def _sc_body(n_classes, rpw,
             tci_hbm, tb_hbm, pred_hbm, sums_hbm, cnts_hbm,
             tci_v, maskf_v, idx_v, gath_v, tbt_v, out_v, sem_g):
    rpad = ((rpw + 127) // 128) * 128
    nchunk = rpad // 128
    stride = n_classes * 4

    wid = lax.axis_index("c") * _NS + lax.axis_index("s")
    base = wid * rpw

    iota = lax.iota(jnp.int32, _L)

    zi = jnp.zeros((_L,), jnp.int32)
    tci_v[pl.ds(rpw - 8, _L)] = zi
    tci_v[pl.ds(rpw + 8, _L)] = zi
    pltpu.sync_copy(tci_hbm.at[pl.ds(base, rpw)], tci_v.at[pl.ds(0, rpw)])

    cnt = jnp.zeros((_L,), jnp.float32)
    for j in range(rpad // _L):
        off = j * _L
        c = tci_v[pl.ds(off, _L)]
        pos = c > 0
        m = jnp.where(pos, 1.0, 0.0).astype(jnp.float32)
        rowl = off + iota
        valid = rowl < rpw
        rowg = base + rowl
        bp = jnp.where(valid, rowg * stride + jnp.where(pos, c, 0) * 4, 0)
        bt = jnp.where(valid, rowg * 4, 0)
        maskf_v[pl.ds(off, _L)] = m
        cnt = cnt + m
        k, q = j // 8, (j % 8) * _L
        for comp in range(4):
            idx_v[comp * nchunk + k, pl.ds(q, _L)] = bp + comp
            idx_v[(4 + comp) * nchunk + k, pl.ds(q, _L)] = bt + comp

    copies = []
    for comp in range(4):
        for k in range(nchunk):
            dst = pl.ds(comp * rpad + k * 128, 128)
            copies.append(pltpu.async_copy(
                pred_hbm.at[idx_v.at[comp * nchunk + k]],
                gath_v.at[dst], sem_g))
            copies.append(pltpu.async_copy(
                tb_hbm.at[idx_v.at[(4 + comp) * nchunk + k]],
                tbt_v.at[dst], sem_g))
    for cp in copies:
        cp.wait()

    def loss_body(j, acc):
        off = j * _L
        m = maskf_v[pl.ds(off, _L)]
        for comp in range(4):
            t = tbt_v[pl.ds(off + comp * rpad, _L)]
            p = gath_v[pl.ds(off + comp * rpad, _L)]
            d = jnp.abs(t - p)
            s = jnp.where(d < 1.0, 0.5 * d * d, d - 0.5)
            acc = acc + s * m
        return acc

    acc = lax.fori_loop(0, rpad // _L, loss_body, jnp.zeros((_L,), jnp.float32))

    out_v[pl.ds(0, _L)] = acc
    out_v[pl.ds(_L, _L)] = cnt
    pltpu.sync_copy(out_v.at[pl.ds(0, _L)], sums_hbm.at[pl.ds(wid * _L, _L)])
    pltpu.sync_copy(out_v.at[pl.ds(_L, _L)], cnts_hbm.at[pl.ds(wid * _L, _L)])


def _combine_body(s_ref, c_ref, o_ref):
    tot = jnp.sum(s_ref[...])
    cnt = jnp.sum(c_ref[...])
    o_ref[...] = jnp.full((1, 1), tot / (cnt * 4.0), jnp.float32)


def kernel(target_bbox, target_class_ids, pred_bbox):
    b, r, n_classes, _ = pred_bbox.shape
    n_rows = b * r
    nw = _NC * _NS
    rpw = n_rows // nw
    rpad = ((rpw + 127) // 128) * 128
    nchunk = rpad // 128

    tci = target_class_ids.reshape(-1).astype(jnp.int32)
    tb = target_bbox.reshape(-1)
    pred = pred_bbox.reshape(-1)

    mesh = plsc.VectorSubcoreMesh(
        core_axis_name="c", subcore_axis_name="s",
        num_cores=_NC, num_subcores=_NS)

    sums, cnts = pl.kernel(
        functools.partial(_sc_body, n_classes, rpw),
        out_type=[
            jax.ShapeDtypeStruct((nw * _L,), jnp.float32),
            jax.ShapeDtypeStruct((nw * _L,), jnp.float32),
        ],
        mesh=mesh,
        scratch_types=[
            pltpu.VMEM((rpad,), jnp.int32),
            pltpu.VMEM((rpad,), jnp.float32),
            pltpu.VMEM((8 * nchunk, 128), jnp.int32),
            pltpu.VMEM((4 * rpad,), jnp.float32),
            pltpu.VMEM((4 * rpad,), jnp.float32),
            pltpu.VMEM((2 * _L,), jnp.float32),
            pltpu.SemaphoreType.DMA,
        ],
    )(tci, tb, pred)

    out = pl.pallas_call(
        _combine_body,
        out_shape=jax.ShapeDtypeStruct((1, 1), jnp.float32),
    )(sums.reshape(nw, _L), cnts.reshape(nw, _L))
    return out.reshape(())


# speedup vs baseline: 17.4489x; 17.4489x over previous
import functools

import jax
import jax.numpy as jnp
from jax import lax
from jax.experimental import pallas as pl
from jax.experimental.pallas import tpu as pltpu
from jax.experimental.pallas import tpu_sc as plsc

_NC = 2
_NS = 16
_L = 16


def _sc_body(n_classes, rpw,
             tci_hbm, tbt_hbm, pred_hbm, sums_hbm, cnts_hbm,
             tci_v, maskf_v, idx_v, gath_v, tbt_v, out_v, sem_g):
    rpad = ((rpw + 127) // 128) * 128
    nchunk = rpad // 128
    stride = n_classes * 4
    n_rows_tot = tci_hbm.size

    wid = lax.axis_index("c") * _NS + lax.axis_index("s")
    base = wid * rpw

    iota = lax.iota(jnp.int32, _L)

    zi = jnp.zeros((_L,), jnp.int32)
    zf = jnp.zeros((_L,), jnp.float32)
    tci_v[pl.ds(rpw - 8, _L)] = zi
    tci_v[pl.ds(rpw + 8, _L)] = zi
    for comp in range(4):
        tbt_v[pl.ds(comp * rpad + rpw - 8, _L)] = zf
        tbt_v[pl.ds(comp * rpad + rpw + 8, _L)] = zf

    tb_copies = [
        pltpu.async_copy(
            tbt_hbm.at[pl.ds((wid * 4 + comp) * rpw, rpw)],
            tbt_v.at[pl.ds(comp * rpad, rpw)], sem_g)
        for comp in range(4)
    ]
    pltpu.sync_copy(tci_hbm.at[pl.ds(base, rpw)], tci_v.at[pl.ds(0, rpw)])

    cnt = jnp.zeros((_L,), jnp.float32)
    for j in range(rpad // _L):
        off = j * _L
        c = tci_v[pl.ds(off, _L)]
        pos = c > 0
        m = jnp.where(pos, 1.0, 0.0).astype(jnp.float32)
        rowl = off + iota
        bp = (base + rowl) * stride + jnp.where(pos, c, 0) * 4
        bp = jnp.where(rowl < rpw, bp, 0)
        maskf_v[pl.ds(off, _L)] = m
        cnt = cnt + m
        k, q = j // 8, (j % 8) * _L
        for comp in range(4):
            idx_v[comp * nchunk + k, pl.ds(q, _L)] = bp + comp

    copies = []
    for comp in range(4):
        for k in range(nchunk):
            copies.append(pltpu.async_copy(
                pred_hbm.at[idx_v.at[comp * nchunk + k]],
                gath_v.at[pl.ds(comp * rpad + k * 128, 128)], sem_g))
    for cp in tb_copies:
        cp.wait()
    for cp in copies:
        cp.wait()

    def loss_body(j, acc):
        off = j * _L
        m = maskf_v[pl.ds(off, _L)]
        for comp in range(4):
            t = tbt_v[pl.ds(off + comp * rpad, _L)]
            p = gath_v[pl.ds(off + comp * rpad, _L)]
            d = jnp.abs(t - p)
            s = jnp.where(d < 1.0, 0.5 * d * d, d - 0.5)
            acc = acc + s * m
        return acc

    acc = lax.fori_loop(0, rpad // _L, loss_body, jnp.zeros((_L,), jnp.float32))

    out_v[pl.ds(0, _L)] = acc
    out_v[pl.ds(_L, _L)] = cnt
    pltpu.sync_copy(out_v.at[pl.ds(0, _L)], sums_hbm.at[pl.ds(wid * _L, _L)])
    pltpu.sync_copy(out_v.at[pl.ds(_L, _L)], cnts_hbm.at[pl.ds(wid * _L, _L)])


def _combine_body(s_ref, c_ref, o_ref):
    tot = jnp.sum(s_ref[...])
    cnt = jnp.sum(c_ref[...])
    o_ref[...] = jnp.full((1, 1), tot / (cnt * 4.0), jnp.float32)


def kernel(target_bbox, target_class_ids, pred_bbox):
    b, r, n_classes, _ = pred_bbox.shape
    n_rows = b * r
    nw = _NC * _NS
    rpw = n_rows // nw
    rpad = ((rpw + 127) // 128) * 128
    nchunk = rpad // 128

    def _flat128(x):
        v = lax.optimization_barrier(x.reshape(x.size // 128, 128))
        return v.reshape(-1)

    tci = _flat128(target_class_ids.astype(jnp.int32))
    tbt = _flat128(lax.optimization_barrier(
        jnp.transpose(target_bbox, (0, 2, 1))))
    p1 = lax.optimization_barrier(pred_bbox.reshape(b, r, n_classes * 4))
    p2 = lax.optimization_barrier(p1.reshape(n_rows, n_classes * 4))
    pred = _flat128(p2)

    mesh = plsc.VectorSubcoreMesh(
        core_axis_name="c", subcore_axis_name="s",
        num_cores=_NC, num_subcores=_NS)

    sums, cnts = pl.kernel(
        functools.partial(_sc_body, n_classes, rpw),
        out_type=[
            jax.ShapeDtypeStruct((nw * _L,), jnp.float32),
            jax.ShapeDtypeStruct((nw * _L,), jnp.float32),
        ],
        mesh=mesh,
        scratch_types=[
            pltpu.VMEM((rpad,), jnp.int32),
            pltpu.VMEM((rpad,), jnp.float32),
            pltpu.VMEM((4 * nchunk, 128), jnp.int32),
            pltpu.VMEM((4 * rpad,), jnp.float32),
            pltpu.VMEM((4 * rpad,), jnp.float32),
            pltpu.VMEM((2 * _L,), jnp.float32),
            pltpu.SemaphoreType.DMA,
        ],
    )(tci, tbt, pred)

    out = pl.pallas_call(
        _combine_body,
        out_shape=jax.ShapeDtypeStruct((1, 1), jnp.float32),
    )(sums.reshape(nw, _L), cnts.reshape(nw, _L))
    return out.reshape(())


# speedup vs baseline: 40.4217x; 2.3166x over previous
import functools

import jax
import jax.numpy as jnp
from jax import lax
from jax.experimental import pallas as pl
from jax.experimental.pallas import tpu as pltpu
from jax.experimental.pallas import tpu_sc as plsc

_NC = 2
_NS = 16
_L = 16


def _sc_body(n_classes, rpw,
             tci_hbm, tbt_hbm, pred_hbm, sums_hbm, cnts_hbm,
             tci_v, maskf_v, idx_v, gath_v, tbt_v, out_v, sem_g):
    rpad = ((rpw + 127) // 128) * 128
    nchunk = rpad // 128
    stride = n_classes * 4
    n_rows_tot = tci_hbm.size

    wid = lax.axis_index("c") * _NS + lax.axis_index("s")
    base = wid * rpw

    iota = lax.iota(jnp.int32, _L)

    zi = jnp.zeros((_L,), jnp.int32)
    zf = jnp.zeros((_L,), jnp.float32)
    tci_v[pl.ds(rpw - 8, _L)] = zi
    tci_v[pl.ds(rpw + 8, _L)] = zi
    for comp in range(4):
        tbt_v[pl.ds(comp * rpad + rpw - 8, _L)] = zf
        tbt_v[pl.ds(comp * rpad + rpw + 8, _L)] = zf

    tb_copies = [
        pltpu.async_copy(
            tbt_hbm.at[pl.ds((wid * 4 + comp) * rpw, rpw)],
            tbt_v.at[pl.ds(comp * rpad, rpw)], sem_g)
        for comp in range(4)
    ]
    pltpu.sync_copy(tci_hbm.at[pl.ds(base, rpw)], tci_v.at[pl.ds(0, rpw)])

    cnt = jnp.zeros((_L,), jnp.float32)
    for j in range(rpad // _L):
        off = j * _L
        c = tci_v[pl.ds(off, _L)]
        pos = c > 0
        m = jnp.where(pos, 1.0, 0.0).astype(jnp.float32)
        rowl = off + iota
        bp = (wid * n_classes + jnp.where(pos, c, 0)) * (4 * rpad) + rowl
        maskf_v[pl.ds(off, _L)] = m
        cnt = cnt + m
        k, q = j // 8, (j % 8) * _L
        for comp in range(4):
            idx_v[comp * nchunk + k, pl.ds(q, _L)] = bp + comp * rpad

    copies = []
    for comp in range(4):
        for k in range(nchunk):
            copies.append(pltpu.async_copy(
                pred_hbm.at[idx_v.at[comp * nchunk + k]],
                gath_v.at[pl.ds(comp * rpad + k * 128, 128)], sem_g))
    for cp in tb_copies:
        cp.wait()
    for cp in copies:
        cp.wait()

    def loss_body(j, acc):
        off = j * _L
        m = maskf_v[pl.ds(off, _L)]
        for comp in range(4):
            t = tbt_v[pl.ds(off + comp * rpad, _L)]
            p = gath_v[pl.ds(off + comp * rpad, _L)]
            d = jnp.abs(t - p)
            s = jnp.where(d < 1.0, 0.5 * d * d, d - 0.5)
            acc = acc + s * m
        return acc

    acc = lax.fori_loop(0, rpad // _L, loss_body, jnp.zeros((_L,), jnp.float32))

    out_v[pl.ds(0, _L)] = acc
    out_v[pl.ds(_L, _L)] = cnt
    pltpu.sync_copy(out_v.at[pl.ds(0, _L)], sums_hbm.at[pl.ds(wid * _L, _L)])
    pltpu.sync_copy(out_v.at[pl.ds(_L, _L)], cnts_hbm.at[pl.ds(wid * _L, _L)])


def _combine_body(s_ref, c_ref, o_ref):
    tot = jnp.sum(s_ref[...])
    cnt = jnp.sum(c_ref[...])
    o_ref[...] = jnp.full((1, 1), tot / (cnt * 4.0), jnp.float32)


def kernel(target_bbox, target_class_ids, pred_bbox):
    b, r, n_classes, _ = pred_bbox.shape
    n_rows = b * r
    nw = _NC * _NS
    rpw = n_rows // nw
    rpad = ((rpw + 127) // 128) * 128
    nchunk = rpad // 128

    def _flat128(x):
        v = lax.optimization_barrier(x.reshape(x.size // 128, 128))
        return v.reshape(-1)

    tci = _flat128(target_class_ids.astype(jnp.int32))
    tbt = _flat128(lax.optimization_barrier(
        jnp.transpose(target_bbox, (0, 2, 1))))
    yp = jnp.pad(pred_bbox, ((0, 0), (0, rpad - rpw), (0, 0), (0, 0)))
    pred = jnp.transpose(yp, (0, 2, 3, 1)).reshape(-1)

    mesh = plsc.VectorSubcoreMesh(
        core_axis_name="c", subcore_axis_name="s",
        num_cores=_NC, num_subcores=_NS)

    sums, cnts = pl.kernel(
        functools.partial(_sc_body, n_classes, rpw),
        out_type=[
            jax.ShapeDtypeStruct((nw * _L,), jnp.float32),
            jax.ShapeDtypeStruct((nw * _L,), jnp.float32),
        ],
        mesh=mesh,
        scratch_types=[
            pltpu.VMEM((rpad,), jnp.int32),
            pltpu.VMEM((rpad,), jnp.float32),
            pltpu.VMEM((4 * nchunk, 128), jnp.int32),
            pltpu.VMEM((4 * rpad,), jnp.float32),
            pltpu.VMEM((4 * rpad,), jnp.float32),
            pltpu.VMEM((2 * _L,), jnp.float32),
            pltpu.SemaphoreType.DMA,
        ],
    )(tci, tbt, pred)

    out = pl.pallas_call(
        _combine_body,
        out_shape=jax.ShapeDtypeStruct((1, 1), jnp.float32),
    )(sums.reshape(nw, _L), cnts.reshape(nw, _L))
    return out.reshape(())


# speedup vs baseline: 63.3297x; 1.5667x over previous
import functools

import jax
import jax.numpy as jnp
from jax import lax
from jax.experimental import pallas as pl
from jax.experimental.pallas import tpu as pltpu
from jax.experimental.pallas import tpu_sc as plsc

_NC = 2
_NS = 16
_L = 16


def _sc_body(n_classes, rpw,
             tci_hbm, tbt_hbm, pred_hbm, sums_hbm, cnts_hbm,
             tci_v, maskf_v, idx_v, gath_v, tbt_v, out_v, sem_g):
    rpad = ((rpw + 127) // 128) * 128
    nchunk = rpad // 128
    stride = n_classes * 4
    n_rows_tot = tci_hbm.size

    wid = lax.axis_index("c") * _NS + lax.axis_index("s")
    base = wid * rpw

    iota = lax.iota(jnp.int32, _L)

    zi = jnp.zeros((_L,), jnp.int32)
    zf = jnp.zeros((_L,), jnp.float32)
    tci_v[pl.ds(rpw - 8, _L)] = zi
    tci_v[pl.ds(rpw + 8, _L)] = zi
    for comp in range(4):
        tbt_v[pl.ds(comp * rpad + rpw - 8, _L)] = zf
        tbt_v[pl.ds(comp * rpad + rpw + 8, _L)] = zf

    tb_copies = [
        pltpu.async_copy(
            tbt_hbm.at[pl.ds((wid * 4 + comp) * rpw, rpw)],
            tbt_v.at[pl.ds(comp * rpad, rpw)], sem_g)
        for comp in range(4)
    ]
    pltpu.sync_copy(tci_hbm.at[pl.ds(base, rpw)], tci_v.at[pl.ds(0, rpw)])

    cnt = jnp.zeros((_L,), jnp.float32)
    for j in range(rpad // _L):
        off = j * _L
        c = tci_v[pl.ds(off, _L)]
        pos = c > 0
        m = jnp.where(pos, 1.0, 0.0).astype(jnp.float32)
        cj = (off >> 7) * 512 + (off & 127)
        bp = (wid * n_classes + jnp.where(pos, c, 0)) * (4 * rpad) + cj + iota
        maskf_v[pl.ds(off, _L)] = m
        cnt = cnt + m
        k, q = j // 8, (j % 8) * _L
        for comp in range(4):
            idx_v[comp * nchunk + k, pl.ds(q, _L)] = bp + comp * 128

    copies = []
    for comp in range(4):
        for k in range(nchunk):
            copies.append(pltpu.async_copy(
                pred_hbm.at[idx_v.at[comp * nchunk + k]],
                gath_v.at[pl.ds(comp * rpad + k * 128, 128)], sem_g))
    for cp in tb_copies:
        cp.wait()
    for cp in copies:
        cp.wait()

    def loss_body(j, acc):
        off = j * _L
        m = maskf_v[pl.ds(off, _L)]
        for comp in range(4):
            t = tbt_v[pl.ds(off + comp * rpad, _L)]
            p = gath_v[pl.ds(off + comp * rpad, _L)]
            d = jnp.abs(t - p)
            s = jnp.where(d < 1.0, 0.5 * d * d, d - 0.5)
            acc = acc + s * m
        return acc

    acc = lax.fori_loop(0, rpad // _L, loss_body, jnp.zeros((_L,), jnp.float32))

    out_v[pl.ds(0, _L)] = acc
    out_v[pl.ds(_L, _L)] = cnt
    pltpu.sync_copy(out_v.at[pl.ds(0, _L)], sums_hbm.at[pl.ds(wid * _L, _L)])
    pltpu.sync_copy(out_v.at[pl.ds(_L, _L)], cnts_hbm.at[pl.ds(wid * _L, _L)])


def _combine_body(s_ref, c_ref, o_ref):
    tot = jnp.sum(s_ref[...])
    cnt = jnp.sum(c_ref[...])
    o_ref[...] = jnp.full((1, 1), tot / (cnt * 4.0), jnp.float32)


def kernel(target_bbox, target_class_ids, pred_bbox):
    b, r, n_classes, _ = pred_bbox.shape
    n_rows = b * r
    nw = _NC * _NS
    rpw = n_rows // nw
    rpad = ((rpw + 127) // 128) * 128
    nchunk = rpad // 128

    def _flat128(x):
        v = lax.optimization_barrier(x.reshape(x.size // 128, 128))
        return v.reshape(-1)

    tci = _flat128(target_class_ids.astype(jnp.int32))
    tbt = _flat128(lax.optimization_barrier(
        jnp.transpose(target_bbox, (0, 2, 1))))
    yp = jnp.pad(pred_bbox, ((0, 0), (0, rpad - rpw), (0, 0), (0, 0)))
    pred = (yp.reshape(b, rpad // 128, 128, n_classes, 4)
            .transpose(0, 3, 1, 4, 2).reshape(-1))

    mesh = plsc.VectorSubcoreMesh(
        core_axis_name="c", subcore_axis_name="s",
        num_cores=_NC, num_subcores=_NS)

    sums, cnts = pl.kernel(
        functools.partial(_sc_body, n_classes, rpw),
        out_type=[
            jax.ShapeDtypeStruct((nw * _L,), jnp.float32),
            jax.ShapeDtypeStruct((nw * _L,), jnp.float32),
        ],
        mesh=mesh,
        scratch_types=[
            pltpu.VMEM((rpad,), jnp.int32),
            pltpu.VMEM((rpad,), jnp.float32),
            pltpu.VMEM((4 * nchunk, 128), jnp.int32),
            pltpu.VMEM((4 * rpad,), jnp.float32),
            pltpu.VMEM((4 * rpad,), jnp.float32),
            pltpu.VMEM((2 * _L,), jnp.float32),
            pltpu.SemaphoreType.DMA,
        ],
    )(tci, tbt, pred)

    out = pl.pallas_call(
        _combine_body,
        out_shape=jax.ShapeDtypeStruct((1, 1), jnp.float32),
    )(sums.reshape(nw, _L), cnts.reshape(nw, _L))
    return out.reshape(())
